# Initial kernel scaffold; baseline (speedup 1.0000x reference)
import jax
import jax.numpy as jnp
from jax.experimental import pallas as pl


def kernel(coordinates, features, ln_g, ln_b, Ws1, bs1, Ws2, bs2, Wm1, bm1, Wm2, bm2, Wm3, bm3, Wm4, bm4, Wi1, bi1, Wi2, bi2, tau):
    raise NotImplementedError("write your pallas kernel here")


# speedup vs baseline: 2.6259x; 2.6259x over previous
import jax
import jax.numpy as jnp
from jax.experimental import pallas as pl

N = 65536
B = 8
M = N // B
FEATURE_DIM = 256
MAX_TOKENS = 128
TOKEN_DIM = 768

ROWS = 1024

_bf = jnp.bfloat16
_f32 = jnp.float32


def _bdot(x, w):
    return jnp.dot(x.astype(_bf), w, preferred_element_type=_f32)


def _mlp_body(cf_ref, feat_ref, lng_ref, lnb_ref,
              ws1_ref, bs1_ref, ws2_ref, bs2_ref,
              w1_ref, b1_ref,
              w2_ref, b2_ref, w3_ref, b3_ref, w4_ref, b4_ref,
              wi1_ref, bi1_ref, wi2_ref, bi2_ref,
              pf_ref, imp_ref, key_ref):
    x4 = cf_ref[...]
    mu = jnp.mean(x4, axis=1, keepdims=True)
    var = jnp.mean((x4 - mu) ** 2, axis=1, keepdims=True)
    cf = (x4 - mu) / jnp.sqrt(var + 1e-5) * lng_ref[...] + lnb_ref[...]

    sp = jnp.maximum(_bdot(cf, ws1_ref[...]) + bs1_ref[...], 0.0)
    sp = _bdot(sp, ws2_ref[...]) + bs2_ref[...]

    cat = jnp.concatenate([feat_ref[...].astype(_bf), sp.astype(_bf)], axis=1)
    h = jnp.maximum(jnp.dot(cat, w1_ref[...], preferred_element_type=_f32)
                    + b1_ref[...], 0.0)
    h = jnp.maximum(_bdot(h, w2_ref[...]) + b2_ref[...], 0.0)
    h = jnp.maximum(_bdot(h, w3_ref[...]) + b3_ref[...], 0.0)
    pf = _bdot(h, w4_ref[...]) + b4_ref[...]
    pf_ref[...] = pf

    t = jnp.maximum(_bdot(pf, wi1_ref[...]) + bi1_ref[...], 0.0)
    imp = _bdot(t, wi2_ref[...]) + bi2_ref[...]
    imp_ref[...] = imp

    bits = jax.lax.bitcast_convert_type(imp, jnp.int32)
    key = bits ^ (jnp.int32(-0x80000000) | (bits >> 31))
    key_ref[...] = key


def _run_mlp(cf4, features, ln_g, ln_b, Ws1, bs1, Ws2, bs2,
             Wm1, bm1, Wm2, bm2, Wm3, bm3, Wm4, bm4, Wi1, bi1, Wi2, bi2):
    grid = N // ROWS
    row_spec = lambda width: pl.BlockSpec((ROWS, width), lambda i: (i, 0))
    full = lambda a: pl.BlockSpec(a.shape, lambda i: (0,) * a.ndim)

    weights = [ln_g[None, :], ln_b[None, :],
               Ws1.T.astype(_bf), bs1[None, :],
               Ws2.T.astype(_bf), bs2[None, :],
               Wm1.T.astype(_bf), bm1[None, :],
               Wm2.T.astype(_bf), bm2[None, :],
               Wm3.T.astype(_bf), bm3[None, :],
               Wm4.T.astype(_bf), bm4[None, :],
               Wi1.T.astype(_bf), bi1[None, :],
               Wi2.T.astype(_bf), bi2[None, :]]

    pf, imp, key = pl.pallas_call(
        _mlp_body,
        grid=(grid,),
        in_specs=[row_spec(4), row_spec(FEATURE_DIM)] + [full(w) for w in weights],
        out_specs=[row_spec(TOKEN_DIM), row_spec(1), row_spec(1)],
        out_shape=[
            jax.ShapeDtypeStruct((N, TOKEN_DIM), jnp.float32),
            jax.ShapeDtypeStruct((N, 1), jnp.float32),
            jax.ShapeDtypeStruct((N, 1), jnp.int32),
        ],
    )(cf4, features, *weights)
    return pf, imp[:, 0], key[:, 0]


def kernel(coordinates, features, ln_g, ln_b, Ws1, bs1, Ws2, bs2,
           Wm1, bm1, Wm2, bm2, Wm3, bm3, Wm4, bm4, Wi1, bi1, Wi2, bi2, tau):
    cf4 = coordinates[:, 1:5]
    pf, imp, key = _run_mlp(cf4, features, ln_g, ln_b, Ws1, bs1, Ws2, bs2,
                            Wm1, bm1, Wm2, bm2, Wm3, bm3, Wm4, bm4,
                            Wi1, bi1, Wi2, bi2)

    sc = imp.reshape(B, M)
    _, ti = jax.lax.top_k(sc, MAX_TOKENS)
    gi = ti + (jnp.arange(B, dtype=ti.dtype) * M)[:, None]
    hf = pf[gi.reshape(-1)].reshape(B, MAX_TOKENS, TOKEN_DIM)
    hc = cf4[gi.reshape(-1)].reshape(B, MAX_TOKENS, 4)
    si = jnp.argsort(hc[:, :, 3], axis=1)
    toks = jnp.take_along_axis(hf, si[:, :, None], axis=1)
    cents = jnp.take_along_axis(hc, si[:, :, None], axis=1)
    masks = jnp.ones((B, MAX_TOKENS), dtype=bool)
    return toks, cents, masks
